# baseline (speedup 1.0000x reference)
import dataclasses
import functools

import jax
import jax.numpy as jnp
from jax import lax
from jax.experimental import pallas as pl
from jax.experimental.pallas import tpu as pltpu
from jax.experimental.pallas import tpu_sc as plsc

NC = 2
NS = 16
L = 16
NW = NC * NS
B = 16384
D = 32
BPW = B // NW
CH = 128
NCHK = BPW // CH
CHG = CH // L
NG = BPW // L

_mesh = plsc.VectorSubcoreMesh(core_axis_name="c", subcore_axis_name="s")

_cp = pltpu.CompilerParams()
if "needs_layout_passes" in pltpu.CompilerParams.__dataclass_fields__:
    _cp = dataclasses.replace(_cp, needs_layout_passes=False)


def _gmf_body(uid_hbm, iid_hbm, uhi_hbm, ihi_hbm, utab_hbm, itab_hbm,
              w_hbm, b_hbm, out_hbm,
              uidx_v, iidx_v, uhi_v, ihi_v, uidx2_v, iidx2_v,
              ub0, ub1, ib0, ib1, wsc_v, b_v, o_v, sem0, sem1):
    wid = lax.axis_index("s") * NC + lax.axis_index("c")
    base = wid * BPW

    pltpu.sync_copy(uid_hbm.at[pl.ds(base, BPW)], uidx_v)
    pltpu.sync_copy(iid_hbm.at[pl.ds(base, BPW)], iidx_v)
    pltpu.sync_copy(uhi_hbm.at[pl.ds(base, BPW)], uhi_v)
    pltpu.sync_copy(ihi_hbm.at[pl.ds(base, BPW)], ihi_v)
    pltpu.sync_copy(w_hbm, wsc_v)
    pltpu.sync_copy(b_hbm, b_v)

    for j in range(NG):
        sl = pl.ds(j * L, L)
        uidx2_v.at[(j, pl.ds(0, L))][...] = uidx_v.at[sl][...]
        iidx2_v.at[(j, pl.ds(0, L))][...] = iidx_v.at[sl][...]

    ubufs = (ub0, ub1)
    ibufs = (ib0, ib1)
    sems = (sem0, sem1)

    def fire(c):
        sl = pl.ds(c * CH, CH)
        p = c % 2
        return (
            pltpu.async_copy(utab_hbm.at[uhi_v.at[sl]], ubufs[p], sems[p]),
            pltpu.async_copy(itab_hbm.at[ihi_v.at[sl]], ibufs[p], sems[p]),
        )

    inflight = fire(0)
    for c in range(NCHK):
        nxt = fire(c + 1) if c + 1 < NCHK else None
        for cp in inflight:
            cp.wait()
        inflight = nxt
        ubp, ibp = ubufs[c % 2], ibufs[c % 2]

        @pl.loop(0, CHG)
        def _(gq):
            g = c * CHG + gq
            uids = uidx2_v.at[(g, pl.ds(0, L))][...]
            iids = iidx2_v.at[(g, pl.ds(0, L))][...]
            lbu = (uids & 3) * D
            lbi = (iids & 3) * D
            rows = gq * L + lax.iota(jnp.int32, L)
            acc = b_v[...]
            for d in range(D):
                uv = plsc.load_gather(ubp, [rows, lbu + d])
                iv = plsc.load_gather(ibp, [rows, lbi + d])
                wv = wsc_v.at[(d, pl.ds(0, L))][...]
                acc = acc + uv * iv * wv
            o_v.at[(g, pl.ds(0, L))][...] = 1.0 / (1.0 + jnp.exp(-acc))

    pltpu.sync_copy(o_v, out_hbm.at[wid])


@functools.partial(
    pl.kernel,
    out_type=jax.ShapeDtypeStruct((NW, NG, L), jnp.float32),
    mesh=_mesh,
    scratch_types=[
        pltpu.VMEM((BPW,), jnp.int32),
        pltpu.VMEM((BPW,), jnp.int32),
        pltpu.VMEM((BPW,), jnp.int32),
        pltpu.VMEM((BPW,), jnp.int32),
        pltpu.VMEM((NG, L), jnp.int32),
        pltpu.VMEM((NG, L), jnp.int32),
        pltpu.VMEM((CH, 128), jnp.float32),
        pltpu.VMEM((CH, 128), jnp.float32),
        pltpu.VMEM((CH, 128), jnp.float32),
        pltpu.VMEM((CH, 128), jnp.float32),
        pltpu.VMEM((D, L), jnp.float32),
        pltpu.VMEM((L,), jnp.float32),
        pltpu.VMEM((NG, L), jnp.float32),
        pltpu.SemaphoreType.DMA,
        pltpu.SemaphoreType.DMA,
    ],
    compiler_params=_cp,
)
def _gmf_sc(*args):
    _gmf_body(*args)


@jax.jit
def kernel(user_ids, item_ids, user_table, item_table, W, b):
    uid = user_ids.astype(jnp.int32)
    iid = item_ids.astype(jnp.int32)
    uhi = jax.lax.shift_right_logical(uid, 2)
    ihi = jax.lax.shift_right_logical(iid, 2)
    utab128 = user_table.reshape(user_table.shape[0] // 4, 128)
    itab128 = item_table.reshape(item_table.shape[0] // 4, 128)
    w_bcast = jnp.broadcast_to(W.reshape(D, 1), (D, L)).astype(jnp.float32)
    b16 = jnp.full((L,), b[0], dtype=jnp.float32)
    out3 = _gmf_sc(uid, iid, uhi, ihi, utab128, itab128, w_bcast, b16)
    return out3.reshape(B)


# speedup vs baseline: 1.4939x; 1.4939x over previous
import dataclasses
import functools

import jax
import jax.numpy as jnp
from jax import lax
from jax.experimental import pallas as pl
from jax.experimental.pallas import tpu as pltpu
from jax.experimental.pallas import tpu_sc as plsc

NC = 2
NS = 16
L = 16
NW = NC * NS
B = 16384
D = 32
BPW = B // NW
CH = 128
NCHK = BPW // CH
CHG = CH // L
NG = BPW // L

_mesh = plsc.VectorSubcoreMesh(core_axis_name="c", subcore_axis_name="s")

_cp = pltpu.CompilerParams()
if "needs_layout_passes" in pltpu.CompilerParams.__dataclass_fields__:
    _cp = dataclasses.replace(_cp, needs_layout_passes=False)


def _gmf_body(uid_hbm, iid_hbm, utab_hbm, itab_hbm, w_hbm, b_hbm, out_hbm,
              usm, ism, uidx_v, iidx_v, ub0, ub1, ib0, ib1, wsc_v, b_v, o_v,
              idsem, sem0, sem1):
    wid = lax.axis_index("s") * NC + lax.axis_index("c")
    base = wid * BPW

    sid = lax.axis_index("s")
    pltpu.async_copy(uid_hbm.at[pl.ds(base, BPW)], uidx_v.at[sid], idsem).wait()
    pltpu.async_copy(iid_hbm.at[pl.ds(base, BPW)], iidx_v.at[sid], idsem).wait()
    pltpu.sync_copy(uidx_v.at[sid], usm)
    pltpu.sync_copy(iidx_v.at[sid], ism)
    pltpu.sync_copy(w_hbm, wsc_v)
    pltpu.sync_copy(b_hbm, b_v)

    ubufs = (ub0, ub1)
    ibufs = (ib0, ib1)
    sems = (sem0, sem1)

    def fire(c):
        p = c % 2
        ubp, ibp, sem = ubufs[p], ibufs[p], sems[p]

        @pl.loop(0, CH)
        def _(r):
            j = c * CH + r
            pltpu.async_copy(utab_hbm.at[usm[j]], ubp.at[r], sem)
            pltpu.async_copy(itab_hbm.at[ism[j]], ibp.at[r], sem)

    def drain(c):
        p = c % 2
        pltpu.make_async_copy(
            utab_hbm.at[pl.ds(0, CH)], ubufs[p], sems[p]).wait()
        pltpu.make_async_copy(
            itab_hbm.at[pl.ds(0, CH)], ibufs[p], sems[p]).wait()

    fire(0)
    for c in range(NCHK):
        if c + 1 < NCHK:
            fire(c + 1)
        drain(c)
        ubp, ibp = ubufs[c % 2], ibufs[c % 2]

        @pl.loop(0, CHG)
        def _(k):
            g = c * CHG + k
            rows = k * L + lax.iota(jnp.int32, L)
            acc = b_v[...]
            for d in range(D):
                didx = jnp.full((L,), d, jnp.int32)
                uv = plsc.load_gather(ubp, [rows, didx])
                iv = plsc.load_gather(ibp, [rows, didx])
                wv = wsc_v.at[(d, pl.ds(0, L))][...]
                acc = acc + uv * iv * wv
            o_v.at[(g, pl.ds(0, L))][...] = 1.0 / (1.0 + jnp.exp(-acc))

    pltpu.sync_copy(o_v, out_hbm.at[wid])


@functools.partial(
    pl.kernel,
    out_type=jax.ShapeDtypeStruct((NW, NG, L), jnp.float32),
    mesh=_mesh,
    scratch_types=[
        pltpu.SMEM((BPW,), jnp.int32),
        pltpu.SMEM((BPW,), jnp.int32),
        pltpu.VMEM_SHARED((NS, BPW), jnp.int32),
        pltpu.VMEM_SHARED((NS, BPW), jnp.int32),
        pltpu.VMEM((CH, D), jnp.float32),
        pltpu.VMEM((CH, D), jnp.float32),
        pltpu.VMEM((CH, D), jnp.float32),
        pltpu.VMEM((CH, D), jnp.float32),
        pltpu.VMEM((D, L), jnp.float32),
        pltpu.VMEM((L,), jnp.float32),
        pltpu.VMEM((NG, L), jnp.float32),
        pltpu.SemaphoreType.DMA,
        pltpu.SemaphoreType.DMA,
        pltpu.SemaphoreType.DMA,
    ],
    compiler_params=_cp,
)
def _gmf_sc(*args):
    _gmf_body(*args)


@jax.jit
def kernel(user_ids, item_ids, user_table, item_table, W, b):
    uid = user_ids.astype(jnp.int32)
    iid = item_ids.astype(jnp.int32)
    w_bcast = jnp.broadcast_to(W.reshape(D, 1), (D, L)).astype(jnp.float32)
    b16 = jnp.full((L,), b[0], dtype=jnp.float32)
    out3 = _gmf_sc(uid, iid, user_table, item_table, w_bcast, b16)
    return out3.reshape(B)
